# baseline (speedup 1.0000x reference)
import jax
import jax.numpy as jnp
from jax.experimental import pallas as pl
from jax.experimental.pallas import tpu as pltpu

B, S, D = 4, 4096, 1024
CHUNK = 64
CPB = 8
G = 2
MAX_LR = 0.2
MIN_DECAY = 0.5
MAX_NORM = 30.0
NORM_EPS = 1e-5

_DNT = (((1,), (1,)), ((), ()))
_DN0 = (((0,), (0,)), ((), ()))


def _hope_kernel(x_ref, m0_ref, ew_ref, eb_ref, aw_ref, ab_ref,
                 gw_ref, gb_ref, w1_ref, w2_ref, out_ref, mfin_ref,
                 s_ref, mbf_ref):
    c = pl.program_id(1)

    @pl.when(c == 0)
    def _init():
        m0 = m0_ref[...]
        s0 = jnp.sum(m0 * m0)
        for i in range(G):
            mfin_ref[i] = m0
            mbf_ref[i] = m0.astype(jnp.bfloat16)
            s_ref[i] = s0

    ew = ew_ref[...]
    aw = aw_ref[...]
    gw = gw_ref[...]
    eb = eb_ref[0]
    ab = ab_ref[0]
    gb = gb_ref[0]
    w1 = w1_ref[...]
    w2 = w2_ref[...]
    ones_rep = jnp.full((8, 128), 0.125, jnp.float32)

    def step(i, k):
        chunk = x_ref[i, k * CHUNK:(k + 1) * CHUNK, :]
        chunk_b = chunk.astype(jnp.bfloat16)
        M = mfin_ref[i]
        Mb = mbf_ref[i]
        S_sc = s_ref[i]

        eta = jnp.mean(jax.nn.sigmoid(
            jnp.sum(chunk * ew, axis=1, keepdims=True) + eb)) * MAX_LR
        alpha = MIN_DECAY + jnp.mean(jax.nn.sigmoid(
            jnp.sum(chunk * aw, axis=1, keepdims=True) + ab)) * (1.0 - MIN_DECAY)

        outputs = jax.lax.dot_general(chunk_b, Mb, _DNT,
                                      preferred_element_type=jnp.float32)

        nrm = jnp.sqrt(jnp.sum(chunk * chunk, axis=1, keepdims=True))
        keys = chunk / jnp.maximum(nrm, NORM_EPS)
        k_mean = jnp.mean(keys, axis=0, keepdims=True)

        h1 = jax.lax.dot_general(outputs.astype(jnp.bfloat16), w1, _DNT,
                                 preferred_element_type=jnp.float32)
        h1 = h1 * jax.nn.sigmoid(h1)
        h = jax.lax.dot_general(h1.astype(jnp.bfloat16), w2, _DNT,
                                preferred_element_type=jnp.float32) + outputs
        out_ref[i, k * CHUNK:(k + 1) * CHUNK, :] = h

        v_t = jnp.mean(h, axis=0, keepdims=True)
        Mk = jax.lax.dot_general(k_mean.astype(jnp.bfloat16), Mb, _DNT,
                                 preferred_element_type=jnp.float32)
        err = v_t - Mk

        gate = jax.nn.sigmoid(jnp.sum(k_mean * gw) + gb)
        gA = gate * alpha + (1.0 - gate)
        gB = gate * eta

        t_cross = jnp.sum(err * Mk)
        r_sq = jnp.sum(err * err) * jnp.sum(k_mean * k_mean)
        fro2 = gA * gA * S_sc + 2.0 * gA * gB * t_cross + gB * gB * r_sq
        scale = jnp.minimum(MAX_NORM / (jnp.sqrt(fro2) + 1e-6), 1.0)

        cA = scale * gA
        cB = scale * gB

        err8 = jnp.broadcast_to(cB * err, (8, D))
        err_rep = jax.lax.dot_general(err8, ones_rep, _DN0,
                                      preferred_element_type=jnp.float32)
        err_full = jnp.concatenate([err_rep] * 8, axis=1)
        m_new = cA * M + err_full * k_mean
        mfin_ref[i] = m_new
        mbf_ref[i] = m_new.astype(jnp.bfloat16)
        s_ref[i] = scale * scale * fro2

    for k in range(CPB):
        for i in range(G):
            step(i, k)


def kernel(x, M_init, eta_w, eta_b, alpha_w, alpha_b, gate_w, gate_b,
           vg_w1, vg_w2):
    bs = CHUNK * CPB
    grid = (B // G, S // bs)

    in_specs = [
        pl.BlockSpec((G, bs, D), lambda g, c: (g, c, 0)),
        pl.BlockSpec((D, D), lambda g, c: (0, 0)),
        pl.BlockSpec((1, D), lambda g, c: (0, 0)),
        pl.BlockSpec(memory_space=pltpu.SMEM),
        pl.BlockSpec((1, D), lambda g, c: (0, 0)),
        pl.BlockSpec(memory_space=pltpu.SMEM),
        pl.BlockSpec((1, D), lambda g, c: (0, 0)),
        pl.BlockSpec(memory_space=pltpu.SMEM),
        pl.BlockSpec((D, D), lambda g, c: (0, 0)),
        pl.BlockSpec((D, D), lambda g, c: (0, 0)),
    ]
    out_specs = [
        pl.BlockSpec((G, bs, D), lambda g, c: (g, c, 0)),
        pl.BlockSpec((G, D, D), lambda g, c: (g, 0, 0)),
    ]
    out_shape = [
        jax.ShapeDtypeStruct((B, S, D), jnp.float32),
        jax.ShapeDtypeStruct((B, D, D), jnp.float32),
    ]

    final_out, m_final = pl.pallas_call(
        _hope_kernel,
        grid=grid,
        in_specs=in_specs,
        out_specs=out_specs,
        out_shape=out_shape,
        scratch_shapes=[pltpu.SMEM((G,), jnp.float32),
                        pltpu.VMEM((G, D, D), jnp.bfloat16)],
        compiler_params=pltpu.CompilerParams(
            dimension_semantics=("parallel", "arbitrary"),
            vmem_limit_bytes=64 * 1024 * 1024,
        ),
    )(x, M_init, eta_w.reshape(1, D), eta_b, alpha_w.reshape(1, D), alpha_b,
      gate_w.reshape(1, D), gate_b, vg_w1.astype(jnp.bfloat16),
      vg_w2.astype(jnp.bfloat16))
    return final_out, m_final


# speedup vs baseline: 1.1299x; 1.1299x over previous
import jax
import jax.numpy as jnp
from jax.experimental import pallas as pl
from jax.experimental.pallas import tpu as pltpu

B, S, D = 4, 4096, 1024
CHUNK = 64
CPB = 8
MAX_LR = 0.2
MIN_DECAY = 0.5
MAX_NORM = 30.0
NORM_EPS = 1e-5

_DNT = (((1,), (1,)), ((), ()))
_DN0 = (((0,), (0,)), ((), ()))
_F32 = jnp.float32
_BF16 = jnp.bfloat16


def _hope_kernel(x_ref, m0_ref, ew_ref, eb_ref, aw_ref, ab_ref,
                 gw_ref, gb_ref, w1_ref, w2_ref, out_ref, mfin_ref,
                 s_ref, mbf_ref):
    c = pl.program_id(1)

    @pl.when(c == 0)
    def _init():
        m0 = m0_ref[...]
        mfin_ref[0] = m0
        mbf_ref[...] = m0.astype(_BF16)
        s_ref[0] = jnp.sum(m0 * m0)

    ew = ew_ref[...]
    aw = aw_ref[...]
    gw = gw_ref[...]
    eb = eb_ref[0]
    ab = ab_ref[0]
    gb = gb_ref[0]
    w1 = w1_ref[...]
    w2 = w2_ref[...]
    ones_rep = jnp.full((8, 128), 0.125, _F32)
    ones_row = jnp.ones((1, 128), _F32)

    def hyper(chunk):
        eta = jnp.mean(jax.nn.sigmoid(
            jnp.sum(chunk * ew, axis=1, keepdims=True) + eb)) * MAX_LR
        alpha = MIN_DECAY + jnp.mean(jax.nn.sigmoid(
            jnp.sum(chunk * aw, axis=1, keepdims=True) + ab)) * (1.0 - MIN_DECAY)
        return eta, alpha

    def kmean_of(chunk):
        nrm = jnp.sqrt(jnp.sum(chunk * chunk, axis=1, keepdims=True))
        keys = chunk / jnp.maximum(nrm, NORM_EPS)
        return jnp.mean(keys, axis=0, keepdims=True)

    def mlp(outputs):
        h1 = jax.lax.dot_general(outputs.astype(_BF16), w1, _DNT,
                                 preferred_element_type=_F32)
        h1 = h1 * jax.nn.sigmoid(h1)
        return jax.lax.dot_general(h1.astype(_BF16), w2, _DNT,
                                   preferred_element_type=_F32) + outputs

    def gates(chunk, k_mean, err, Mk, S_sc):
        eta, alpha = hyper(chunk)
        gate = jax.nn.sigmoid(jnp.sum(k_mean * gw) + gb)
        gA = gate * alpha + (1.0 - gate)
        gB = gate * eta
        t_cross = jnp.sum(err * Mk)
        r_sq = jnp.sum(err * err) * jnp.sum(k_mean * k_mean)
        fro2 = gA * gA * S_sc + 2.0 * gA * gB * t_cross + gB * gB * r_sq
        scale = jnp.minimum(MAX_NORM / (jnp.sqrt(fro2) + 1e-6), 1.0)
        return scale * gA, scale * gB, scale * scale * fro2

    def col_bcast(row):
        r8 = jnp.broadcast_to(row, (8, D))
        rep = jax.lax.dot_general(r8, ones_rep, _DN0,
                                  preferred_element_type=_F32)
        return jnp.concatenate([rep] * 8, axis=1)

    for p in range(CPB // 2):
        k0 = 2 * p
        ch01 = x_ref[0, k0 * CHUNK:(k0 + 2) * CHUNK, :]
        ch0 = ch01[:CHUNK]
        ch1 = ch01[CHUNK:]
        M = mfin_ref[0]
        Mb = mbf_ref[...]
        S_sc = s_ref[0]

        k0m = kmean_of(ch0)
        k1m = kmean_of(ch1)
        kp = jnp.concatenate([k0m, k1m], axis=0)

        base = jax.lax.dot_general(ch01.astype(_BF16), Mb, _DNT,
                                   preferred_element_type=_F32)
        mk_base = jax.lax.dot_general(kp.astype(_BF16), Mb, _DNT,
                                      preferred_element_type=_F32)

        out0 = base[:CHUNK]
        h0 = mlp(out0)
        out_ref[0, k0 * CHUNK:(k0 + 1) * CHUNK, :] = h0
        v0 = jnp.mean(h0, axis=0, keepdims=True)
        mk0 = mk_base[:1]
        err0 = v0 - mk0
        cA0, cB0, S0 = gates(ch0, k0m, err0, mk0, S_sc)

        kq0 = cB0 * k0m
        q = jnp.sum(ch1 * kq0, axis=1, keepdims=True)
        q_rep = jax.lax.dot_general(q, ones_row, (((1,), (0,)), ((), ())),
                                    preferred_element_type=_F32)
        q_full = jnp.concatenate([q_rep] * 8, axis=1)
        out1 = cA0 * base[CHUNK:] + q_full * err0
        h1 = mlp(out1)
        out_ref[0, (k0 + 1) * CHUNK:(k0 + 2) * CHUNK, :] = h1
        v1 = jnp.mean(h1, axis=0, keepdims=True)
        s_dot = jnp.sum(k1m * kq0)
        mk1 = cA0 * mk_base[1:] + s_dot * err0
        err1 = v1 - mk1
        cA1, cB1, S1 = gates(ch1, k1m, err1, mk1, S0)

        e0_full = col_bcast((cA1 * cB0) * err0)
        e1_full = col_bcast(cB1 * err1)
        m_new = (cA1 * cA0) * M + e0_full * k0m + e1_full * k1m
        mfin_ref[0] = m_new
        mbf_ref[...] = m_new.astype(_BF16)
        s_ref[0] = S1


def kernel(x, M_init, eta_w, eta_b, alpha_w, alpha_b, gate_w, gate_b,
           vg_w1, vg_w2):
    bs = CHUNK * CPB
    grid = (B, S // bs)

    in_specs = [
        pl.BlockSpec((1, bs, D), lambda b, c: (b, c, 0)),
        pl.BlockSpec((D, D), lambda b, c: (0, 0)),
        pl.BlockSpec((1, D), lambda b, c: (0, 0)),
        pl.BlockSpec(memory_space=pltpu.SMEM),
        pl.BlockSpec((1, D), lambda b, c: (0, 0)),
        pl.BlockSpec(memory_space=pltpu.SMEM),
        pl.BlockSpec((1, D), lambda b, c: (0, 0)),
        pl.BlockSpec(memory_space=pltpu.SMEM),
        pl.BlockSpec((D, D), lambda b, c: (0, 0)),
        pl.BlockSpec((D, D), lambda b, c: (0, 0)),
    ]
    out_specs = [
        pl.BlockSpec((1, bs, D), lambda b, c: (b, c, 0)),
        pl.BlockSpec((1, D, D), lambda b, c: (b, 0, 0)),
    ]
    out_shape = [
        jax.ShapeDtypeStruct((B, S, D), jnp.float32),
        jax.ShapeDtypeStruct((B, D, D), jnp.float32),
    ]

    final_out, m_final = pl.pallas_call(
        _hope_kernel,
        grid=grid,
        in_specs=in_specs,
        out_specs=out_specs,
        out_shape=out_shape,
        scratch_shapes=[pltpu.SMEM((1,), jnp.float32),
                        pltpu.VMEM((D, D), jnp.bfloat16)],
        compiler_params=pltpu.CompilerParams(
            dimension_semantics=("parallel", "arbitrary"),
            vmem_limit_bytes=64 * 1024 * 1024,
        ),
    )(x, M_init, eta_w.reshape(1, D), eta_b, alpha_w.reshape(1, D), alpha_b,
      gate_w.reshape(1, D), gate_b, vg_w1.astype(jnp.bfloat16),
      vg_w2.astype(jnp.bfloat16))
    return final_out, m_final


# speedup vs baseline: 1.2340x; 1.0921x over previous
import jax
import jax.numpy as jnp
from jax.experimental import pallas as pl
from jax.experimental.pallas import tpu as pltpu

B, S, D = 4, 4096, 1024
CHUNK = 64
CPB = 8
GRP = 4
MAX_LR = 0.2
MIN_DECAY = 0.5
MAX_NORM = 30.0
NORM_EPS = 1e-5

_DNT = (((1,), (1,)), ((), ()))
_DN0 = (((0,), (0,)), ((), ()))
_DNS = (((1,), (0,)), ((), ()))
_F32 = jnp.float32
_BF16 = jnp.bfloat16


def _hope_kernel(x_ref, m0_ref, ew_ref, eb_ref, aw_ref, ab_ref,
                 gw_ref, gb_ref, w1_ref, w2_ref, out_ref, mfin_ref,
                 s_ref, mbf_ref):
    c = pl.program_id(1)

    @pl.when(c == 0)
    def _init():
        m0 = m0_ref[...]
        mfin_ref[0] = m0
        mbf_ref[...] = m0.astype(_BF16)
        s_ref[0] = jnp.sum(m0 * m0)

    ew = ew_ref[...]
    aw = aw_ref[...]
    gw = gw_ref[...]
    eb = eb_ref[0]
    ab = ab_ref[0]
    gb = gb_ref[0]
    w1 = w1_ref[...]
    w2 = w2_ref[...]
    ones_rep = jnp.full((8, 128), 0.125, _F32)

    def hyper(chunk):
        eta = jnp.mean(jax.nn.sigmoid(
            jnp.sum(chunk * ew, axis=1, keepdims=True) + eb)) * MAX_LR
        alpha = MIN_DECAY + jnp.mean(jax.nn.sigmoid(
            jnp.sum(chunk * aw, axis=1, keepdims=True) + ab)) * (1.0 - MIN_DECAY)
        return eta, alpha

    def kmean_of(chunk):
        nrm = jnp.sqrt(jnp.sum(chunk * chunk, axis=1, keepdims=True))
        keys = chunk / jnp.maximum(nrm, NORM_EPS)
        return jnp.mean(keys, axis=0, keepdims=True)

    def mlp(outputs):
        h1 = jax.lax.dot_general(outputs.astype(_BF16), w1, _DNT,
                                 preferred_element_type=_F32)
        h1 = h1 * jax.nn.sigmoid(h1)
        return jax.lax.dot_general(h1.astype(_BF16), w2, _DNT,
                                   preferred_element_type=_F32) + outputs

    def gates(chunk, k_mean, err, Mk, S_sc):
        eta, alpha = hyper(chunk)
        gate = jax.nn.sigmoid(jnp.sum(k_mean * gw) + gb)
        gA = gate * alpha + (1.0 - gate)
        gB = gate * eta
        t_cross = jnp.sum(err * Mk)
        r_sq = jnp.sum(err * err) * jnp.sum(k_mean * k_mean)
        fro2 = gA * gA * S_sc + 2.0 * gA * gB * t_cross + gB * gB * r_sq
        scale = jnp.minimum(MAX_NORM / (jnp.sqrt(fro2) + 1e-6), 1.0)
        return scale * gA, scale * gB, scale * scale * fro2

    def col_bcast(row):
        r8 = jnp.broadcast_to(row, (8, D))
        rep = jax.lax.dot_general(r8, ones_rep, _DN0,
                                  preferred_element_type=_F32)
        return jnp.concatenate([rep] * 8, axis=1)

    for g in range(CPB // GRP):
        k0 = g * GRP
        chg = x_ref[0, k0 * CHUNK:(k0 + GRP) * CHUNK, :]
        chunks = [chg[j * CHUNK:(j + 1) * CHUNK] for j in range(GRP)]
        M = mfin_ref[0]
        Mb = mbf_ref[...]
        S_sc = s_ref[0]

        kms = [kmean_of(ch) for ch in chunks]
        kp = jnp.concatenate(kms, axis=0)

        base = jax.lax.dot_general(chg.astype(_BF16), Mb, _DNT,
                                   preferred_element_type=_F32)
        mkb = jax.lax.dot_general(kp.astype(_BF16), Mb, _DNT,
                                  preferred_element_type=_F32)

        P = 1.0
        gcoef = []
        errs = []
        upds = []
        for j in range(GRP):
            ch = chunks[j]
            kj = kms[j]
            bj = base[j * CHUNK:(j + 1) * CHUNK]
            if j == 0:
                outj = bj
                mkj = mkb[:1]
            else:
                kt = jnp.concatenate(
                    [gcoef[t] * kms[t] for t in range(j)], axis=0)
                et = jnp.concatenate(errs, axis=0)
                q = jax.lax.dot_general(ch, kt, _DNT,
                                        preferred_element_type=_F32)
                corr = jax.lax.dot_general(q, et, _DNS,
                                           preferred_element_type=_F32)
                outj = P * bj + corr
                mk_corr = sum(
                    (gcoef[t] * jnp.sum(kj * kms[t])) * errs[t]
                    for t in range(j))
                mkj = P * mkb[j:j + 1] + mk_corr

            hj = mlp(outj)
            out_ref[0, (k0 + j) * CHUNK:(k0 + j + 1) * CHUNK, :] = hj
            vj = jnp.mean(hj, axis=0, keepdims=True)
            errj = vj - mkj
            cA, cB, S_sc = gates(ch, kj, errj, mkj, S_sc)

            P = P * cA
            gcoef = [gc * cA for gc in gcoef] + [cB]
            errs = errs + [errj]

        m_new = P * M
        for t in range(GRP):
            ef = col_bcast(gcoef[t] * errs[t])
            m_new = m_new + ef * kms[t]
        mfin_ref[0] = m_new
        mbf_ref[...] = m_new.astype(_BF16)
        s_ref[0] = S_sc


def kernel(x, M_init, eta_w, eta_b, alpha_w, alpha_b, gate_w, gate_b,
           vg_w1, vg_w2):
    bs = CHUNK * CPB
    grid = (B, S // bs)

    in_specs = [
        pl.BlockSpec((1, bs, D), lambda b, c: (b, c, 0)),
        pl.BlockSpec((D, D), lambda b, c: (0, 0)),
        pl.BlockSpec((1, D), lambda b, c: (0, 0)),
        pl.BlockSpec(memory_space=pltpu.SMEM),
        pl.BlockSpec((1, D), lambda b, c: (0, 0)),
        pl.BlockSpec(memory_space=pltpu.SMEM),
        pl.BlockSpec((1, D), lambda b, c: (0, 0)),
        pl.BlockSpec(memory_space=pltpu.SMEM),
        pl.BlockSpec((D, D), lambda b, c: (0, 0)),
        pl.BlockSpec((D, D), lambda b, c: (0, 0)),
    ]
    out_specs = [
        pl.BlockSpec((1, bs, D), lambda b, c: (b, c, 0)),
        pl.BlockSpec((1, D, D), lambda b, c: (b, 0, 0)),
    ]
    out_shape = [
        jax.ShapeDtypeStruct((B, S, D), jnp.float32),
        jax.ShapeDtypeStruct((B, D, D), jnp.float32),
    ]

    final_out, m_final = pl.pallas_call(
        _hope_kernel,
        grid=grid,
        in_specs=in_specs,
        out_specs=out_specs,
        out_shape=out_shape,
        scratch_shapes=[pltpu.SMEM((1,), jnp.float32),
                        pltpu.VMEM((D, D), jnp.bfloat16)],
        compiler_params=pltpu.CompilerParams(
            dimension_semantics=("parallel", "arbitrary"),
            vmem_limit_bytes=64 * 1024 * 1024,
        ),
    )(x, M_init, eta_w.reshape(1, D), eta_b, alpha_w.reshape(1, D), alpha_b,
      gate_w.reshape(1, D), gate_b, vg_w1.astype(jnp.bfloat16),
      vg_w2.astype(jnp.bfloat16))
    return final_out, m_final


# speedup vs baseline: 1.2465x; 1.0101x over previous
import jax
import jax.numpy as jnp
from jax.experimental import pallas as pl
from jax.experimental.pallas import tpu as pltpu

B, S, D = 4, 4096, 1024
CHUNK = 64
CPB = 8
GRP = 8
MAX_LR = 0.2
MIN_DECAY = 0.5
MAX_NORM = 30.0
NORM_EPS = 1e-5

_DNT = (((1,), (1,)), ((), ()))
_DN0 = (((0,), (0,)), ((), ()))
_DNS = (((1,), (0,)), ((), ()))
_F32 = jnp.float32
_BF16 = jnp.bfloat16


def _hope_kernel(x_ref, m0_ref, ew_ref, eb_ref, aw_ref, ab_ref,
                 gw_ref, gb_ref, w1_ref, w2_ref, out_ref, mfin_ref,
                 s_ref, mbf_ref):
    c = pl.program_id(1)

    @pl.when(c == 0)
    def _init():
        m0 = m0_ref[...]
        mfin_ref[0] = m0
        mbf_ref[...] = m0.astype(_BF16)
        s_ref[0] = jnp.sum(m0 * m0)

    ew = ew_ref[...]
    aw = aw_ref[...]
    gw = gw_ref[...]
    eb = eb_ref[0]
    ab = ab_ref[0]
    gb = gb_ref[0]
    w1 = w1_ref[...]
    w2 = w2_ref[...]
    ones_rep = jnp.full((8, 128), 0.125, _F32)

    def hyper(chunk):
        eta = jnp.mean(jax.nn.sigmoid(
            jnp.sum(chunk * ew, axis=1, keepdims=True) + eb)) * MAX_LR
        alpha = MIN_DECAY + jnp.mean(jax.nn.sigmoid(
            jnp.sum(chunk * aw, axis=1, keepdims=True) + ab)) * (1.0 - MIN_DECAY)
        return eta, alpha

    def kmean_of(chunk):
        nrm = jnp.sqrt(jnp.sum(chunk * chunk, axis=1, keepdims=True))
        keys = chunk / jnp.maximum(nrm, NORM_EPS)
        return jnp.mean(keys, axis=0, keepdims=True)

    def mlp(outputs):
        h1 = jax.lax.dot_general(outputs.astype(_BF16), w1, _DNT,
                                 preferred_element_type=_F32)
        h1 = h1 * jax.nn.sigmoid(h1)
        return jax.lax.dot_general(h1.astype(_BF16), w2, _DNT,
                                   preferred_element_type=_F32) + outputs

    def gates(chunk, k_mean, err, Mk, S_sc):
        eta, alpha = hyper(chunk)
        gate = jax.nn.sigmoid(jnp.sum(k_mean * gw) + gb)
        gA = gate * alpha + (1.0 - gate)
        gB = gate * eta
        t_cross = jnp.sum(err * Mk)
        r_sq = jnp.sum(err * err) * jnp.sum(k_mean * k_mean)
        fro2 = gA * gA * S_sc + 2.0 * gA * gB * t_cross + gB * gB * r_sq
        scale = jnp.minimum(MAX_NORM / (jnp.sqrt(fro2) + 1e-6), 1.0)
        return scale * gA, scale * gB, scale * scale * fro2

    def col_bcast(row):
        r8 = jnp.broadcast_to(row, (8, D))
        rep = jax.lax.dot_general(r8, ones_rep, _DN0,
                                  preferred_element_type=_F32)
        return jnp.concatenate([rep] * 8, axis=1)

    for g in range(CPB // GRP):
        k0 = g * GRP
        chg = x_ref[0, k0 * CHUNK:(k0 + GRP) * CHUNK, :]
        chunks = [chg[j * CHUNK:(j + 1) * CHUNK] for j in range(GRP)]
        M = mfin_ref[0]
        Mb = mbf_ref[...]
        S_sc = s_ref[0]

        kms = [kmean_of(ch) for ch in chunks]
        kp = jnp.concatenate(kms, axis=0)

        base = jax.lax.dot_general(chg.astype(_BF16), Mb, _DNT,
                                   preferred_element_type=_F32)
        mkb = jax.lax.dot_general(kp.astype(_BF16), Mb, _DNT,
                                  preferred_element_type=_F32)

        P = 1.0
        gcoef = []
        errs = []
        upds = []
        for j in range(GRP):
            ch = chunks[j]
            kj = kms[j]
            bj = base[j * CHUNK:(j + 1) * CHUNK]
            if j == 0:
                outj = bj
                mkj = mkb[:1]
            else:
                kt = jnp.concatenate(
                    [gcoef[t] * kms[t] for t in range(j)], axis=0)
                et = jnp.concatenate(errs, axis=0)
                q = jax.lax.dot_general(ch, kt, _DNT,
                                        preferred_element_type=_F32)
                corr = jax.lax.dot_general(q, et, _DNS,
                                           preferred_element_type=_F32)
                outj = P * bj + corr
                mk_corr = sum(
                    (gcoef[t] * jnp.sum(kj * kms[t])) * errs[t]
                    for t in range(j))
                mkj = P * mkb[j:j + 1] + mk_corr

            hj = mlp(outj)
            out_ref[0, (k0 + j) * CHUNK:(k0 + j + 1) * CHUNK, :] = hj
            vj = jnp.mean(hj, axis=0, keepdims=True)
            errj = vj - mkj
            cA, cB, S_sc = gates(ch, kj, errj, mkj, S_sc)

            P = P * cA
            gcoef = [gc * cA for gc in gcoef] + [cB]
            errs = errs + [errj]

        m_new = P * M
        for t in range(GRP):
            ef = col_bcast(gcoef[t] * errs[t])
            m_new = m_new + ef * kms[t]
        mfin_ref[0] = m_new
        mbf_ref[...] = m_new.astype(_BF16)
        s_ref[0] = S_sc


def kernel(x, M_init, eta_w, eta_b, alpha_w, alpha_b, gate_w, gate_b,
           vg_w1, vg_w2):
    bs = CHUNK * CPB
    grid = (B, S // bs)

    in_specs = [
        pl.BlockSpec((1, bs, D), lambda b, c: (b, c, 0)),
        pl.BlockSpec((D, D), lambda b, c: (0, 0)),
        pl.BlockSpec((1, D), lambda b, c: (0, 0)),
        pl.BlockSpec(memory_space=pltpu.SMEM),
        pl.BlockSpec((1, D), lambda b, c: (0, 0)),
        pl.BlockSpec(memory_space=pltpu.SMEM),
        pl.BlockSpec((1, D), lambda b, c: (0, 0)),
        pl.BlockSpec(memory_space=pltpu.SMEM),
        pl.BlockSpec((D, D), lambda b, c: (0, 0)),
        pl.BlockSpec((D, D), lambda b, c: (0, 0)),
    ]
    out_specs = [
        pl.BlockSpec((1, bs, D), lambda b, c: (b, c, 0)),
        pl.BlockSpec((1, D, D), lambda b, c: (b, 0, 0)),
    ]
    out_shape = [
        jax.ShapeDtypeStruct((B, S, D), jnp.float32),
        jax.ShapeDtypeStruct((B, D, D), jnp.float32),
    ]

    final_out, m_final = pl.pallas_call(
        _hope_kernel,
        grid=grid,
        in_specs=in_specs,
        out_specs=out_specs,
        out_shape=out_shape,
        scratch_shapes=[pltpu.SMEM((1,), jnp.float32),
                        pltpu.VMEM((D, D), jnp.bfloat16)],
        compiler_params=pltpu.CompilerParams(
            dimension_semantics=("parallel", "arbitrary"),
            vmem_limit_bytes=64 * 1024 * 1024,
        ),
    )(x, M_init, eta_w.reshape(1, D), eta_b, alpha_w.reshape(1, D), alpha_b,
      gate_w.reshape(1, D), gate_b, vg_w1.astype(jnp.bfloat16),
      vg_w2.astype(jnp.bfloat16))
    return final_out, m_final


# speedup vs baseline: 1.3666x; 1.0964x over previous
import jax
import jax.numpy as jnp
from jax.experimental import pallas as pl
from jax.experimental.pallas import tpu as pltpu

B, S, D = 4, 4096, 1024
CHUNK = 64
CPB = 8
GRP = 8
MAX_LR = 0.2
MIN_DECAY = 0.5
MAX_NORM = 30.0
NORM_EPS = 1e-5

_DNT = (((1,), (1,)), ((), ()))
_DN0 = (((0,), (0,)), ((), ()))
_DNS = (((1,), (0,)), ((), ()))
_F32 = jnp.float32
_BF16 = jnp.bfloat16


def _hope_kernel(x_ref, m0_ref, ew_ref, eb_ref, aw_ref, ab_ref,
                 gw_ref, gb_ref, w1_ref, w2_ref, out_ref, mfin_ref,
                 s_ref, mbf_ref):
    c = pl.program_id(1)

    @pl.when(c == 0)
    def _init():
        m0 = m0_ref[...]
        mfin_ref[0] = m0
        mbf_ref[...] = m0.astype(_BF16)
        s_ref[0] = jnp.sum(m0 * m0)

    ew = ew_ref[...]
    aw = aw_ref[...]
    gw = gw_ref[...]
    eb = eb_ref[0]
    ab = ab_ref[0]
    gb = gb_ref[0]
    w1 = w1_ref[...]
    w2 = w2_ref[...]
    ones_rep = jnp.full((8, 128), 0.125, _F32)

    def hyper(chunk):
        eta = jnp.mean(jax.nn.sigmoid(
            jnp.sum(chunk * ew, axis=1, keepdims=True) + eb)) * MAX_LR
        alpha = MIN_DECAY + jnp.mean(jax.nn.sigmoid(
            jnp.sum(chunk * aw, axis=1, keepdims=True) + ab)) * (1.0 - MIN_DECAY)
        return eta, alpha

    def kmean_of(chunk):
        nrm = jnp.sqrt(jnp.sum(chunk * chunk, axis=1, keepdims=True))
        keys = chunk / jnp.maximum(nrm, NORM_EPS)
        return jnp.mean(keys, axis=0, keepdims=True)

    def gates(chunk, k_mean, err, Mk, S_sc):
        eta, alpha = hyper(chunk)
        gate = jax.nn.sigmoid(jnp.sum(k_mean * gw) + gb)
        gA = gate * alpha + (1.0 - gate)
        gB = gate * eta
        t_cross = jnp.sum(err * Mk)
        r_sq = jnp.sum(err * err) * jnp.sum(k_mean * k_mean)
        fro2 = gA * gA * S_sc + 2.0 * gA * gB * t_cross + gB * gB * r_sq
        scale = jnp.minimum(MAX_NORM / (jnp.sqrt(fro2) + 1e-6), 1.0)
        return scale * gA, scale * gB, scale * scale * fro2

    def col_bcast(row):
        r8 = jnp.broadcast_to(row, (8, D))
        rep = jax.lax.dot_general(r8, ones_rep, _DN0,
                                  preferred_element_type=_F32)
        return jnp.concatenate([rep] * 8, axis=1)

    for g in range(CPB // GRP):
        k0 = g * GRP
        chg = x_ref[0, k0 * CHUNK:(k0 + GRP) * CHUNK, :]
        chunks = [chg[j * CHUNK:(j + 1) * CHUNK] for j in range(GRP)]
        M = mfin_ref[0]
        Mb = mbf_ref[...]
        S_sc = s_ref[0]

        kms = [kmean_of(ch) for ch in chunks]
        kp = jnp.concatenate(kms, axis=0)

        base = jax.lax.dot_general(chg.astype(_BF16), Mb, _DNT,
                                   preferred_element_type=_F32)
        mkb = jax.lax.dot_general(kp.astype(_BF16), Mb, _DNT,
                                  preferred_element_type=_F32)
        bw = jax.lax.dot_general(base.astype(_BF16), w1, _DNT,
                                 preferred_element_type=_F32)

        P = 1.0
        gcoef = []
        errs = []
        ew1s = []
        h1s = []
        outs = []
        for j in range(GRP):
            ch = chunks[j]
            kj = kms[j]
            bj = base[j * CHUNK:(j + 1) * CHUNK]
            bm = jnp.mean(bj, axis=0, keepdims=True)
            if j == 0:
                zj = bw[:CHUNK]
                outj = bj
                om = bm
                mkj = mkb[:1]
            else:
                kt = jnp.concatenate(
                    [gcoef[t] * kms[t] for t in range(j)], axis=0)
                et = jnp.concatenate(errs, axis=0)
                ew1t = jnp.concatenate(ew1s, axis=0)
                q = jax.lax.dot_general(ch, kt, _DNT,
                                        preferred_element_type=_F32)
                qm = jnp.mean(q, axis=0, keepdims=True)
                zj = P * bw[j * CHUNK:(j + 1) * CHUNK] + \
                    jax.lax.dot_general(q, ew1t, _DNS,
                                        preferred_element_type=_F32)
                outj = P * bj + jax.lax.dot_general(
                    q, et, _DNS, preferred_element_type=_F32)
                om = P * bm + jax.lax.dot_general(
                    qm, et, _DNS, preferred_element_type=_F32)
                mk_corr = sum(
                    (gcoef[t] * jnp.sum(kj * kms[t])) * errs[t]
                    for t in range(j))
                mkj = P * mkb[j:j + 1] + mk_corr

            h1 = zj * jax.nn.sigmoid(zj)
            h1s.append(h1)
            outs.append(outj)
            h1m = jnp.mean(h1, axis=0, keepdims=True)
            vj = jax.lax.dot_general(h1m.astype(_BF16), w2, _DNT,
                                     preferred_element_type=_F32) + om
            errj = vj - mkj
            cA, cB, S_sc = gates(ch, kj, errj, mkj, S_sc)

            P = P * cA
            gcoef = [gc * cA for gc in gcoef] + [cB]
            errs = errs + [errj]
            ew1s = ew1s + [jax.lax.dot_general(
                errj.astype(_BF16), w1, _DNT, preferred_element_type=_F32)]

        h1g = jnp.concatenate(h1s, axis=0)
        outg = jnp.concatenate(outs, axis=0)
        hg = jax.lax.dot_general(h1g.astype(_BF16), w2, _DNT,
                                 preferred_element_type=_F32) + outg
        out_ref[0, k0 * CHUNK:(k0 + GRP) * CHUNK, :] = hg

        m_new = P * M
        for t in range(GRP):
            ef = col_bcast(gcoef[t] * errs[t])
            m_new = m_new + ef * kms[t]
        mfin_ref[0] = m_new
        mbf_ref[...] = m_new.astype(_BF16)
        s_ref[0] = S_sc


def kernel(x, M_init, eta_w, eta_b, alpha_w, alpha_b, gate_w, gate_b,
           vg_w1, vg_w2):
    bs = CHUNK * CPB
    grid = (B, S // bs)

    in_specs = [
        pl.BlockSpec((1, bs, D), lambda b, c: (b, c, 0)),
        pl.BlockSpec((D, D), lambda b, c: (0, 0)),
        pl.BlockSpec((1, D), lambda b, c: (0, 0)),
        pl.BlockSpec(memory_space=pltpu.SMEM),
        pl.BlockSpec((1, D), lambda b, c: (0, 0)),
        pl.BlockSpec(memory_space=pltpu.SMEM),
        pl.BlockSpec((1, D), lambda b, c: (0, 0)),
        pl.BlockSpec(memory_space=pltpu.SMEM),
        pl.BlockSpec((D, D), lambda b, c: (0, 0)),
        pl.BlockSpec((D, D), lambda b, c: (0, 0)),
    ]
    out_specs = [
        pl.BlockSpec((1, bs, D), lambda b, c: (b, c, 0)),
        pl.BlockSpec((1, D, D), lambda b, c: (b, 0, 0)),
    ]
    out_shape = [
        jax.ShapeDtypeStruct((B, S, D), jnp.float32),
        jax.ShapeDtypeStruct((B, D, D), jnp.float32),
    ]

    final_out, m_final = pl.pallas_call(
        _hope_kernel,
        grid=grid,
        in_specs=in_specs,
        out_specs=out_specs,
        out_shape=out_shape,
        scratch_shapes=[pltpu.SMEM((1,), jnp.float32),
                        pltpu.VMEM((D, D), jnp.bfloat16)],
        compiler_params=pltpu.CompilerParams(
            dimension_semantics=("parallel", "arbitrary"),
            vmem_limit_bytes=64 * 1024 * 1024,
        ),
    )(x, M_init, eta_w.reshape(1, D), eta_b, alpha_w.reshape(1, D), alpha_b,
      gate_w.reshape(1, D), gate_b, vg_w1.astype(jnp.bfloat16),
      vg_w2.astype(jnp.bfloat16))
    return final_out, m_final


# speedup vs baseline: 1.5941x; 1.1664x over previous
import jax
import jax.numpy as jnp
from jax.experimental import pallas as pl
from jax.experimental.pallas import tpu as pltpu

B, S, D = 4, 4096, 1024
CHUNK = 64
CPB = 16
GRP = 8
MAX_LR = 0.2
MIN_DECAY = 0.5
MAX_NORM = 30.0
NORM_EPS = 1e-5

_DNT = (((1,), (1,)), ((), ()))
_DN0 = (((0,), (0,)), ((), ()))
_DNS = (((1,), (0,)), ((), ()))
_F32 = jnp.float32
_BF16 = jnp.bfloat16


def _hope_kernel(x_ref, m0_ref, ew_ref, eb_ref, aw_ref, ab_ref,
                 gw_ref, gb_ref, w1_ref, w2_ref, out_ref, mfin_ref,
                 s_ref, mbf_ref):
    c = pl.program_id(1)

    @pl.when(c == 0)
    def _init():
        m0 = m0_ref[...]
        mfin_ref[0] = m0
        mbf_ref[...] = m0.astype(_BF16)
        s_ref[0] = jnp.sum(m0 * m0)

    ew = ew_ref[...]
    aw = aw_ref[...]
    gw = gw_ref[...]
    eb = eb_ref[0]
    ab = ab_ref[0]
    gb = gb_ref[0]
    w1 = w1_ref[...]
    w2 = w2_ref[...]
    def hyper(chunk):
        eta = jnp.mean(jax.nn.sigmoid(
            jnp.sum(chunk * ew, axis=1, keepdims=True) + eb)) * MAX_LR
        alpha = MIN_DECAY + jnp.mean(jax.nn.sigmoid(
            jnp.sum(chunk * aw, axis=1, keepdims=True) + ab)) * (1.0 - MIN_DECAY)
        return eta, alpha

    def kmean_of(chunk):
        nrm = jnp.sqrt(jnp.sum(chunk * chunk, axis=1, keepdims=True))
        keys = chunk / jnp.maximum(nrm, NORM_EPS)
        return jnp.mean(keys, axis=0, keepdims=True)

    def gates(chunk, k_mean, err, Mk, S_sc):
        eta, alpha = hyper(chunk)
        gate = jax.nn.sigmoid(jnp.sum(k_mean * gw) + gb)
        gA = gate * alpha + (1.0 - gate)
        gB = gate * eta
        t_cross = jnp.sum(err * Mk)
        r_sq = jnp.sum(err * err) * jnp.sum(k_mean * k_mean)
        fro2 = gA * gA * S_sc + 2.0 * gA * gB * t_cross + gB * gB * r_sq
        scale = jnp.minimum(MAX_NORM / (jnp.sqrt(fro2) + 1e-6), 1.0)
        return scale * gA, scale * gB, scale * scale * fro2

    for g in range(CPB // GRP):
        k0 = g * GRP
        chg = x_ref[0, k0 * CHUNK:(k0 + GRP) * CHUNK, :]
        chunks = [chg[j * CHUNK:(j + 1) * CHUNK] for j in range(GRP)]
        M = mfin_ref[0]
        Mb = mbf_ref[...]
        S_sc = s_ref[0]

        kms = [kmean_of(ch) for ch in chunks]
        kp = jnp.concatenate(kms, axis=0)

        base = jax.lax.dot_general(chg.astype(_BF16), Mb, _DNT,
                                   preferred_element_type=_F32)
        mkb = jax.lax.dot_general(kp.astype(_BF16), Mb, _DNT,
                                  preferred_element_type=_F32)
        bw = jax.lax.dot_general(base.astype(_BF16), w1, _DNT,
                                 preferred_element_type=_F32)

        P = 1.0
        gcoef = []
        errs = []
        ew1s = []
        h1s = []
        outs = []
        for j in range(GRP):
            ch = chunks[j]
            kj = kms[j]
            bj = base[j * CHUNK:(j + 1) * CHUNK]
            bm = jnp.mean(bj, axis=0, keepdims=True)
            if j == 0:
                zj = bw[:CHUNK]
                outj = bj
                om = bm
                mkj = mkb[:1]
            else:
                kt = jnp.concatenate(
                    [gcoef[t] * kms[t] for t in range(j)], axis=0)
                et = jnp.concatenate(errs, axis=0)
                ew1t = jnp.concatenate(ew1s, axis=0)
                q = jax.lax.dot_general(ch, kt, _DNT,
                                        preferred_element_type=_F32)
                qm = jnp.mean(q, axis=0, keepdims=True)
                zj = P * bw[j * CHUNK:(j + 1) * CHUNK] + \
                    jax.lax.dot_general(q, ew1t, _DNS,
                                        preferred_element_type=_F32)
                outj = P * bj + jax.lax.dot_general(
                    q, et, _DNS, preferred_element_type=_F32)
                om = P * bm + jax.lax.dot_general(
                    qm, et, _DNS, preferred_element_type=_F32)
                mk_corr = sum(
                    (gcoef[t] * jnp.sum(kj * kms[t])) * errs[t]
                    for t in range(j))
                mkj = P * mkb[j:j + 1] + mk_corr

            h1 = zj * jax.nn.sigmoid(zj)
            h1s.append(h1)
            outs.append(outj)
            h1m = jnp.mean(h1, axis=0, keepdims=True)
            vj = jax.lax.dot_general(h1m.astype(_BF16), w2, _DNT,
                                     preferred_element_type=_F32) + om
            errj = vj - mkj
            cA, cB, S_sc = gates(ch, kj, errj, mkj, S_sc)

            P = P * cA
            gcoef = [gc * cA for gc in gcoef] + [cB]
            errs = errs + [errj]
            ew1s = ew1s + [jax.lax.dot_general(
                errj.astype(_BF16), w1, _DNT, preferred_element_type=_F32)]

        h1g = jnp.concatenate(h1s, axis=0)
        outg = jnp.concatenate(outs, axis=0)
        hg = jax.lax.dot_general(h1g.astype(_BF16), w2, _DNT,
                                 preferred_element_type=_F32) + outg
        out_ref[0, k0 * CHUNK:(k0 + GRP) * CHUNK, :] = hg

        eg = jnp.concatenate([gcoef[t] * errs[t] for t in range(GRP)], axis=0)
        upd = jax.lax.dot_general(eg, kp, _DN0,
                                  preferred_element_type=_F32)
        m_new = P * M + upd
        mfin_ref[0] = m_new
        mbf_ref[...] = m_new.astype(_BF16)
        s_ref[0] = S_sc


def kernel(x, M_init, eta_w, eta_b, alpha_w, alpha_b, gate_w, gate_b,
           vg_w1, vg_w2):
    bs = CHUNK * CPB
    grid = (B, S // bs)

    in_specs = [
        pl.BlockSpec((1, bs, D), lambda b, c: (b, c, 0)),
        pl.BlockSpec((D, D), lambda b, c: (0, 0)),
        pl.BlockSpec((1, D), lambda b, c: (0, 0)),
        pl.BlockSpec(memory_space=pltpu.SMEM),
        pl.BlockSpec((1, D), lambda b, c: (0, 0)),
        pl.BlockSpec(memory_space=pltpu.SMEM),
        pl.BlockSpec((1, D), lambda b, c: (0, 0)),
        pl.BlockSpec(memory_space=pltpu.SMEM),
        pl.BlockSpec((D, D), lambda b, c: (0, 0)),
        pl.BlockSpec((D, D), lambda b, c: (0, 0)),
    ]
    out_specs = [
        pl.BlockSpec((1, bs, D), lambda b, c: (b, c, 0)),
        pl.BlockSpec((1, D, D), lambda b, c: (b, 0, 0)),
    ]
    out_shape = [
        jax.ShapeDtypeStruct((B, S, D), jnp.float32),
        jax.ShapeDtypeStruct((B, D, D), jnp.float32),
    ]

    final_out, m_final = pl.pallas_call(
        _hope_kernel,
        grid=grid,
        in_specs=in_specs,
        out_specs=out_specs,
        out_shape=out_shape,
        scratch_shapes=[pltpu.SMEM((1,), jnp.float32),
                        pltpu.VMEM((D, D), jnp.bfloat16)],
        compiler_params=pltpu.CompilerParams(
            dimension_semantics=("parallel", "arbitrary"),
            vmem_limit_bytes=64 * 1024 * 1024,
        ),
    )(x, M_init, eta_w.reshape(1, D), eta_b, alpha_w.reshape(1, D), alpha_b,
      gate_w.reshape(1, D), gate_b, vg_w1.astype(jnp.bfloat16),
      vg_w2.astype(jnp.bfloat16))
    return final_out, m_final


# speedup vs baseline: 1.5979x; 1.0024x over previous
import jax
import jax.numpy as jnp
from jax.experimental import pallas as pl
from jax.experimental.pallas import tpu as pltpu

B, S, D = 4, 4096, 1024
CHUNK = 64
CPB = 16
GRP = 16
MAX_LR = 0.2
MIN_DECAY = 0.5
MAX_NORM = 30.0
NORM_EPS = 1e-5

_DNT = (((1,), (1,)), ((), ()))
_DN0 = (((0,), (0,)), ((), ()))
_DNS = (((1,), (0,)), ((), ()))
_F32 = jnp.float32
_BF16 = jnp.bfloat16


def _hope_kernel(x_ref, m0_ref, ew_ref, eb_ref, aw_ref, ab_ref,
                 gw_ref, gb_ref, w1_ref, w2_ref, out_ref, mfin_ref,
                 s_ref, mbf_ref):
    c = pl.program_id(1)

    @pl.when(c == 0)
    def _init():
        m0 = m0_ref[...]
        mfin_ref[0] = m0
        mbf_ref[...] = m0.astype(_BF16)
        s_ref[0] = jnp.sum(m0 * m0)

    ew = ew_ref[...]
    aw = aw_ref[...]
    gw = gw_ref[...]
    eb = eb_ref[0]
    ab = ab_ref[0]
    gb = gb_ref[0]
    w1 = w1_ref[...]
    w2 = w2_ref[...]
    def hyper(chunk):
        eta = jnp.mean(jax.nn.sigmoid(
            jnp.sum(chunk * ew, axis=1, keepdims=True) + eb)) * MAX_LR
        alpha = MIN_DECAY + jnp.mean(jax.nn.sigmoid(
            jnp.sum(chunk * aw, axis=1, keepdims=True) + ab)) * (1.0 - MIN_DECAY)
        return eta, alpha

    def kmean_of(chunk):
        nrm = jnp.sqrt(jnp.sum(chunk * chunk, axis=1, keepdims=True))
        keys = chunk / jnp.maximum(nrm, NORM_EPS)
        return jnp.mean(keys, axis=0, keepdims=True)

    def gates(chunk, k_mean, err, Mk, S_sc):
        eta, alpha = hyper(chunk)
        gate = jax.nn.sigmoid(jnp.sum(k_mean * gw) + gb)
        gA = gate * alpha + (1.0 - gate)
        gB = gate * eta
        t_cross = jnp.sum(err * Mk)
        r_sq = jnp.sum(err * err) * jnp.sum(k_mean * k_mean)
        fro2 = gA * gA * S_sc + 2.0 * gA * gB * t_cross + gB * gB * r_sq
        scale = jnp.minimum(MAX_NORM / (jnp.sqrt(fro2) + 1e-6), 1.0)
        return scale * gA, scale * gB, scale * scale * fro2

    for g in range(CPB // GRP):
        k0 = g * GRP
        chg = x_ref[0, k0 * CHUNK:(k0 + GRP) * CHUNK, :]
        chunks = [chg[j * CHUNK:(j + 1) * CHUNK] for j in range(GRP)]
        M = mfin_ref[0]
        Mb = mbf_ref[...]
        S_sc = s_ref[0]

        kms = [kmean_of(ch) for ch in chunks]
        kp = jnp.concatenate(kms, axis=0)

        base = jax.lax.dot_general(chg.astype(_BF16), Mb, _DNT,
                                   preferred_element_type=_F32)
        mkb = jax.lax.dot_general(kp.astype(_BF16), Mb, _DNT,
                                  preferred_element_type=_F32)
        bw = jax.lax.dot_general(base.astype(_BF16), w1, _DNT,
                                 preferred_element_type=_F32)

        P = 1.0
        gcoef = []
        errs = []
        ew1s = []
        h1s = []
        outs = []
        for j in range(GRP):
            ch = chunks[j]
            kj = kms[j]
            bj = base[j * CHUNK:(j + 1) * CHUNK]
            bm = jnp.mean(bj, axis=0, keepdims=True)
            if j == 0:
                zj = bw[:CHUNK]
                outj = bj
                om = bm
                mkj = mkb[:1]
            else:
                kt = jnp.concatenate(
                    [gcoef[t] * kms[t] for t in range(j)], axis=0)
                et = jnp.concatenate(errs, axis=0)
                ew1t = jnp.concatenate(ew1s, axis=0)
                q = jax.lax.dot_general(ch, kt, _DNT,
                                        preferred_element_type=_F32)
                qm = jnp.mean(q, axis=0, keepdims=True)
                zj = P * bw[j * CHUNK:(j + 1) * CHUNK] + \
                    jax.lax.dot_general(q, ew1t, _DNS,
                                        preferred_element_type=_F32)
                outj = P * bj + jax.lax.dot_general(
                    q, et, _DNS, preferred_element_type=_F32)
                om = P * bm + jax.lax.dot_general(
                    qm, et, _DNS, preferred_element_type=_F32)
                mk_corr = sum(
                    (gcoef[t] * jnp.sum(kj * kms[t])) * errs[t]
                    for t in range(j))
                mkj = P * mkb[j:j + 1] + mk_corr

            h1 = zj * jax.nn.sigmoid(zj)
            h1s.append(h1)
            outs.append(outj)
            h1m = jnp.mean(h1, axis=0, keepdims=True)
            vj = jax.lax.dot_general(h1m.astype(_BF16), w2, _DNT,
                                     preferred_element_type=_F32) + om
            errj = vj - mkj
            cA, cB, S_sc = gates(ch, kj, errj, mkj, S_sc)

            P = P * cA
            gcoef = [gc * cA for gc in gcoef] + [cB]
            errs = errs + [errj]
            ew1s = ew1s + [jax.lax.dot_general(
                errj.astype(_BF16), w1, _DNT, preferred_element_type=_F32)]

        h1g = jnp.concatenate(h1s, axis=0)
        outg = jnp.concatenate(outs, axis=0)
        hg = jax.lax.dot_general(h1g.astype(_BF16), w2, _DNT,
                                 preferred_element_type=_F32) + outg
        out_ref[0, k0 * CHUNK:(k0 + GRP) * CHUNK, :] = hg

        eg = jnp.concatenate([gcoef[t] * errs[t] for t in range(GRP)], axis=0)
        upd = jax.lax.dot_general(eg, kp, _DN0,
                                  preferred_element_type=_F32)
        m_new = P * M + upd
        mfin_ref[0] = m_new
        mbf_ref[...] = m_new.astype(_BF16)
        s_ref[0] = S_sc


def kernel(x, M_init, eta_w, eta_b, alpha_w, alpha_b, gate_w, gate_b,
           vg_w1, vg_w2):
    bs = CHUNK * CPB
    grid = (B, S // bs)

    in_specs = [
        pl.BlockSpec((1, bs, D), lambda b, c: (b, c, 0)),
        pl.BlockSpec((D, D), lambda b, c: (0, 0)),
        pl.BlockSpec((1, D), lambda b, c: (0, 0)),
        pl.BlockSpec(memory_space=pltpu.SMEM),
        pl.BlockSpec((1, D), lambda b, c: (0, 0)),
        pl.BlockSpec(memory_space=pltpu.SMEM),
        pl.BlockSpec((1, D), lambda b, c: (0, 0)),
        pl.BlockSpec(memory_space=pltpu.SMEM),
        pl.BlockSpec((D, D), lambda b, c: (0, 0)),
        pl.BlockSpec((D, D), lambda b, c: (0, 0)),
    ]
    out_specs = [
        pl.BlockSpec((1, bs, D), lambda b, c: (b, c, 0)),
        pl.BlockSpec((1, D, D), lambda b, c: (b, 0, 0)),
    ]
    out_shape = [
        jax.ShapeDtypeStruct((B, S, D), jnp.float32),
        jax.ShapeDtypeStruct((B, D, D), jnp.float32),
    ]

    final_out, m_final = pl.pallas_call(
        _hope_kernel,
        grid=grid,
        in_specs=in_specs,
        out_specs=out_specs,
        out_shape=out_shape,
        scratch_shapes=[pltpu.SMEM((1,), jnp.float32),
                        pltpu.VMEM((D, D), jnp.bfloat16)],
        compiler_params=pltpu.CompilerParams(
            dimension_semantics=("parallel", "arbitrary"),
            vmem_limit_bytes=64 * 1024 * 1024,
        ),
    )(x, M_init, eta_w.reshape(1, D), eta_b, alpha_w.reshape(1, D), alpha_b,
      gate_w.reshape(1, D), gate_b, vg_w1.astype(jnp.bfloat16),
      vg_w2.astype(jnp.bfloat16))
    return final_out, m_final
